# Initial kernel scaffold; baseline (speedup 1.0000x reference)
import jax
import jax.numpy as jnp
from jax.experimental import pallas as pl


def kernel(x, edge_index, batch, W1, a1_src, a1_dst, b1, W2, a2_src, a2_dst, b2, Wfc, bfc):
    raise NotImplementedError("write your pallas kernel here")


# speedup vs baseline: 6.9744x; 6.9744x over previous
import jax
import jax.numpy as jnp
from jax.experimental import pallas as pl

_N = 100000
_E = 1600000
_EP = _E + _N
_HID = 64
_HEADS = 2
_F = _HEADS * _HID
_G = 64

_NODE_BLK = 1000
_EDGE_BLK = 2000


def _proj_kernel(x_ref, w_ref, asrc_ref, adst_ref, h_ref, als_ref, ald_ref):
    h = jnp.dot(x_ref[...], w_ref[...], preferred_element_type=jnp.float32)
    h_ref[...] = h
    hs = h * asrc_ref[...]
    hd = h * adst_ref[...]
    als_ref[...] = jnp.concatenate(
        [hs[:, :_HID].sum(axis=1, keepdims=True),
         hs[:, _HID:].sum(axis=1, keepdims=True)], axis=1)
    ald_ref[...] = jnp.concatenate(
        [hd[:, :_HID].sum(axis=1, keepdims=True),
         hd[:, _HID:].sum(axis=1, keepdims=True)], axis=1)


def _node_project(x, W, a_src_flat, a_dst_flat):
    in_dim = x.shape[1]
    return pl.pallas_call(
        _proj_kernel,
        grid=(_N // _NODE_BLK,),
        in_specs=[
            pl.BlockSpec((_NODE_BLK, in_dim), lambda i: (i, 0)),
            pl.BlockSpec((in_dim, _F), lambda i: (0, 0)),
            pl.BlockSpec((1, _F), lambda i: (0, 0)),
            pl.BlockSpec((1, _F), lambda i: (0, 0)),
        ],
        out_specs=[
            pl.BlockSpec((_NODE_BLK, _F), lambda i: (i, 0)),
            pl.BlockSpec((_NODE_BLK, _HEADS), lambda i: (i, 0)),
            pl.BlockSpec((_NODE_BLK, _HEADS), lambda i: (i, 0)),
        ],
        out_shape=[
            jax.ShapeDtypeStruct((_N, _F), jnp.float32),
            jax.ShapeDtypeStruct((_N, _HEADS), jnp.float32),
            jax.ShapeDtypeStruct((_N, _HEADS), jnp.float32),
        ],
    )(x, W, a_src_flat, a_dst_flat)


def _logit_kernel(es_ref, ed_ref, e_ref):
    e = es_ref[...] + ed_ref[...]
    e_ref[...] = jnp.where(e >= 0.0, e, 0.2 * e)


def _edge_logits(es, ed):
    return pl.pallas_call(
        _logit_kernel,
        grid=(_EP // _EDGE_BLK,),
        in_specs=[pl.BlockSpec((_EDGE_BLK, _HEADS), lambda i: (i, 0))] * 2,
        out_specs=pl.BlockSpec((_EDGE_BLK, _HEADS), lambda i: (i, 0)),
        out_shape=jax.ShapeDtypeStruct((_EP, _HEADS), jnp.float32),
    )(es, ed)


def _exp_kernel(e_ref, emax_ref, ee_ref):
    ee_ref[...] = jnp.exp(e_ref[...] - emax_ref[...])


def _edge_exp(e, emax_d):
    return pl.pallas_call(
        _exp_kernel,
        grid=(_EP // _EDGE_BLK,),
        in_specs=[pl.BlockSpec((_EDGE_BLK, _HEADS), lambda i: (i, 0))] * 2,
        out_specs=pl.BlockSpec((_EDGE_BLK, _HEADS), lambda i: (i, 0)),
        out_shape=jax.ShapeDtypeStruct((_EP, _HEADS), jnp.float32),
    )(e, emax_d)


def _msg_kernel(h_ref, ee_ref, den_ref, msg_ref):
    alpha = ee_ref[...] / (den_ref[...] + 1e-16)
    h = h_ref[...]
    msg_ref[...] = jnp.concatenate(
        [h[:, :_HID] * alpha[:, 0:1], h[:, _HID:] * alpha[:, 1:2]], axis=1)


def _edge_messages(h_src, ee, den_d):
    return pl.pallas_call(
        _msg_kernel,
        grid=(_EP // _EDGE_BLK,),
        in_specs=[
            pl.BlockSpec((_EDGE_BLK, _F), lambda i: (i, 0)),
            pl.BlockSpec((_EDGE_BLK, _HEADS), lambda i: (i, 0)),
            pl.BlockSpec((_EDGE_BLK, _HEADS), lambda i: (i, 0)),
        ],
        out_specs=pl.BlockSpec((_EDGE_BLK, _F), lambda i: (i, 0)),
        out_shape=jax.ShapeDtypeStruct((_EP, _F), jnp.float32),
    )(h_src, ee, den_d)


def _elu_kernel(agg_ref, b_ref, out_ref):
    v = agg_ref[...] + b_ref[...]
    out_ref[...] = jnp.where(v > 0.0, v, jnp.exp(jnp.minimum(v, 0.0)) - 1.0)


def _node_elu(agg, b_flat):
    return pl.pallas_call(
        _elu_kernel,
        grid=(_N // _NODE_BLK,),
        in_specs=[
            pl.BlockSpec((_NODE_BLK, _F), lambda i: (i, 0)),
            pl.BlockSpec((1, _F), lambda i: (0, 0)),
        ],
        out_specs=pl.BlockSpec((_NODE_BLK, _F), lambda i: (i, 0)),
        out_shape=jax.ShapeDtypeStruct((_N, _F), jnp.float32),
    )(agg, b_flat)


def _fc_kernel(p_ref, w_ref, b_ref, o_ref):
    o_ref[...] = jnp.dot(p_ref[...], w_ref[...],
                         preferred_element_type=jnp.float32) + b_ref[...]


def _gat_layer(h_in, src, dst, W, a_s, a_d, b):
    h, al_s, al_d = _node_project(h_in, W,
                                  a_s.reshape(1, _F), a_d.reshape(1, _F))
    e = _edge_logits(al_s[src], al_d[dst])
    emax = jax.ops.segment_max(e, dst, num_segments=_N)
    ee = _edge_exp(e, emax[dst])
    denom = jax.ops.segment_sum(ee, dst, num_segments=_N)
    msg = _edge_messages(h[src], ee, denom[dst])
    agg = jax.ops.segment_sum(msg, dst, num_segments=_N)
    return _node_elu(agg, b.reshape(1, _F))


def kernel(x, edge_index, batch, W1, a1_src, a1_dst, b1, W2, a2_src, a2_dst, b2, Wfc, bfc):
    loop = jnp.arange(_N, dtype=edge_index.dtype)
    src = jnp.concatenate([edge_index[0], loop])
    dst = jnp.concatenate([edge_index[1], loop])

    h = _gat_layer(x, src, dst, W1, a1_src, a1_dst, b1)
    h = _gat_layer(h, src, dst, W2, a2_src, a2_dst, b2)

    sums = jax.ops.segment_sum(h, batch, num_segments=_G)
    counts = jax.ops.segment_sum(jnp.ones((_N,), jnp.float32), batch,
                                 num_segments=_G)
    pooled = sums / jnp.maximum(counts, 1.0)[:, None]

    return pl.pallas_call(
        _fc_kernel,
        in_specs=[
            pl.BlockSpec((_G, _F), lambda: (0, 0)),
            pl.BlockSpec((_F, _G), lambda: (0, 0)),
            pl.BlockSpec((1, _G), lambda: (0, 0)),
        ],
        out_specs=pl.BlockSpec((_G, _G), lambda: (0, 0)),
        out_shape=jax.ShapeDtypeStruct((_G, _G), jnp.float32),
    )(pooled, Wfc.T, bfc.reshape(1, _G))
